# baseline (speedup 1.0000x reference)
import functools

import jax
import jax.numpy as jnp
from jax import lax
from jax.experimental import pallas as pl
from jax.experimental.pallas import tpu as pltpu
from jax.experimental.pallas import tpu_sc as plsc

N = 10000
E = 320000
D = 128
EPS = 1e-5

NW = 32
EW = E // NW
CH = 2000
VPC = CH // 16
NCH = EW // CH
ZR = 16

_MESH = plsc.VectorSubcoreMesh(core_axis_name="c", subcore_axis_name="s")
_PARAMS = pltpu.CompilerParams(needs_layout_passes=False)


def _bcast_lane(vec, j):
    return jnp.take_along_axis(vec, jnp.full((16,), j, jnp.int32), axis=0)


def _make_attention_kernel(with_att):
    lacc = 6 * N if with_att else N

    def body(src_h, dst_h, att_h, asrc_h, adst_h, pv_h,
             w_h, part_h,
             asrc_v, adst_v, pv_v, acc_v, srcb, dstb, attb, wb):
        c = lax.axis_index("c")
        s = lax.axis_index("s")
        wid = s * 2 + c
        pltpu.sync_copy(asrc_h, asrc_v)
        pltpu.sync_copy(adst_h, adst_v)
        pltpu.sync_copy(pv_h, pv_v)
        pv = pv_v[...]
        v0 = _bcast_lane(pv, 0)
        v1 = _bcast_lane(pv, 1)
        v2 = _bcast_lane(pv, 2)
        v3 = _bcast_lane(pv, 3)
        zero16 = jnp.zeros((16,), jnp.float32)
        ones16 = jnp.ones((16,), jnp.float32)

        def zacc(i, carry):
            acc_v[pl.ds(i * 16, 16)] = zero16
            return carry
        lax.fori_loop(0, lacc // 16, zacc, 0)

        ebase = wid * EW

        def chunk(ci, carry):
            base = ebase + ci * CH
            pltpu.sync_copy(src_h.at[pl.ds(base, CH)], srcb)
            pltpu.sync_copy(dst_h.at[pl.ds(base, CH)], dstb)
            for ch in range(4):
                pltpu.sync_copy(att_h.at[pl.ds(ch * E + base, CH)],
                                attb.at[pl.ds(ch * CH, CH)])

            def ebody(v, carry2):
                o = v * 16
                sv = srcb[pl.ds(o, 16)]
                dv = dstb[pl.ds(o, 16)]
                a0 = attb[pl.ds(0 * CH + o, 16)]
                a1 = attb[pl.ds(1 * CH + o, 16)]
                a2 = attb[pl.ds(2 * CH + o, 16)]
                a3 = attb[pl.ds(3 * CH + o, 16)]
                ae = a0 * v0 + a1 * v1 + a2 * v2 + a3 * v3
                asg = plsc.load_gather(asrc_v, [sv])
                adg = plsc.load_gather(adst_v, [dv])
                al = asg + adg + ae
                al = jnp.where(al > 0, al, al * 0.2)
                wv = jnp.exp(al)
                wb[pl.ds(o, 16)] = wv
                if with_att:
                    plsc.addupdate_scatter(acc_v, [dv], ones16)
                    plsc.addupdate_scatter(acc_v, [dv + N], a0)
                    plsc.addupdate_scatter(acc_v, [dv + 2 * N], a1)
                    plsc.addupdate_scatter(acc_v, [dv + 3 * N], a2)
                    plsc.addupdate_scatter(acc_v, [dv + 4 * N], a3)
                    plsc.addupdate_scatter(acc_v, [dv + 5 * N], wv)
                else:
                    plsc.addupdate_scatter(acc_v, [dv], wv)
                return carry2
            lax.fori_loop(0, VPC, ebody, 0)
            pltpu.sync_copy(wb, w_h.at[pl.ds(base, CH)])
            return carry
        lax.fori_loop(0, NCH, chunk, 0)

        pltpu.sync_copy(acc_v, part_h.at[pl.ds(wid * lacc, lacc)])

    return pl.kernel(
        body,
        out_type=[jax.ShapeDtypeStruct((E,), jnp.float32),
                  jax.ShapeDtypeStruct((NW * lacc,), jnp.float32)],
        mesh=_MESH,
        scratch_types=[
            pltpu.VMEM((N,), jnp.float32),
            pltpu.VMEM((N,), jnp.float32),
            pltpu.VMEM((16,), jnp.float32),
            pltpu.VMEM((lacc,), jnp.float32),
            pltpu.VMEM((CH,), jnp.int32),
            pltpu.VMEM((CH,), jnp.int32),
            pltpu.VMEM((4 * CH,), jnp.float32),
            pltpu.VMEM((CH,), jnp.float32),
        ],
        compiler_params=_PARAMS,
    )


NB = 5
GROUPS = VPC // NB
ACH = EW
AVPC = ACH // 16
AGROUPS = AVPC // NB


def _agg_body(src_h, dst_h, w_h, h_h,
              outp_h,
              srcb, dstb, wb, gb, sb, shared_acc,
              gs0, gs1, gs2, gs3, gs4, ss0, ss1, ss2, ss3, ss4):
    gsems = (gs0, gs1, gs2, gs3, gs4)
    ssems = (ss0, ss1, ss2, ss3, ss4)
    c = lax.axis_index("c")
    s = lax.axis_index("s")
    wid = s * 2 + c
    zero16 = jnp.zeros((16,), jnp.float32)
    for r in range(ZR):
        for k in range(D // 16):
            sb[0, r, pl.ds(k * 16, 16)] = zero16
    zbase = s * 624

    def zsh(i, carry):
        pltpu.sync_copy(sb.at[0], shared_acc.at[pl.ds(zbase + i * ZR, ZR)])
        return carry
    lax.fori_loop(0, 624 // ZR, zsh, 0)

    @pl.when(s == 15)
    def _():
        pltpu.sync_copy(sb.at[0], shared_acc.at[pl.ds(624 * 16, ZR)])
    plsc.subcore_barrier()

    base = wid * EW
    pltpu.sync_copy(src_h.at[pl.ds(base, ACH)], srcb)
    pltpu.sync_copy(dst_h.at[pl.ds(base, ACH)], dstb)
    pltpu.sync_copy(w_h.at[pl.ds(base, ACH)], wb)
    for b in range(NB):
        svb = srcb[pl.ds(b * 16, 16)]
        pltpu.make_async_copy(h_h.at[svb], gb.at[b], gsems[b]).start()

    def group(g, carry2):
        for b in range(NB):
            v = g * NB + b
            o = v * 16
            dv = dstb[pl.ds(o, 16)]
            wv = wb[pl.ds(o, 16)]

            @pl.when(g > 0)
            def _wait_scatter():
                pltpu.make_async_copy(
                    sb.at[b], shared_acc.at[dv], ssems[b]).wait()
            pltpu.make_async_copy(h_h.at[dv], gb.at[b], gsems[b]).wait()
            for j in range(16):
                wj = _bcast_lane(wv, j)
                for k in range(D // 16):
                    sb[b, j, pl.ds(k * 16, 16)] = (
                        gb[b, j, pl.ds(k * 16, 16)] * wj)
            pltpu.make_async_copy(
                sb.at[b], shared_acc.at[dv], ssems[b]).start(add=True)

            @pl.when(g < AGROUPS - 1)
            def _issue_gather():
                svn = srcb[pl.ds((v + NB) * 16, 16)]
                pltpu.make_async_copy(h_h.at[svn], gb.at[b], gsems[b]).start()
        return carry2
    lax.fori_loop(0, AGROUPS, group, 0)
    dv0 = dstb[pl.ds(0, 16)]
    for b in range(NB):
        pltpu.make_async_copy(sb.at[b], shared_acc.at[dv0], ssems[b]).wait()

    plsc.subcore_barrier()

    @pl.when(s == 0)
    def _():
        pltpu.sync_copy(shared_acc, outp_h.at[c])


_agg_kernel = pl.kernel(
    _agg_body,
    out_type=[jax.ShapeDtypeStruct((2, N, D), jnp.float32)],
    mesh=_MESH,
    scratch_types=[
        pltpu.VMEM((ACH,), jnp.int32),
        pltpu.VMEM((ACH,), jnp.int32),
        pltpu.VMEM((ACH,), jnp.float32),
        pltpu.VMEM((NB, 16, D), jnp.float32),
        pltpu.VMEM((NB, 16, D), jnp.float32),
        pltpu.VMEM_SHARED((N, D), jnp.float32),
    ] + [pltpu.SemaphoreType.DMA] * 10,
    compiler_params=_PARAMS,
)

_att_l1 = _make_attention_kernel(True)
_att_l2 = _make_attention_kernel(False)


NB_TC = 2000
_GRID = N // NB_TC
_INV = float(1.0 / (1.0 + EPS) ** 0.5)


def _mm_body(x_ref, w_ref, as_ref, ad_ref, h_ref, asrc_ref, adst_ref):
    h = x_ref[...] @ w_ref[...]
    h_ref[...] = h
    asrc_ref[...] = (h * as_ref[...]).sum(axis=1).reshape(NB_TC, 1)
    adst_ref[...] = (h * ad_ref[...]).sum(axis=1).reshape(NB_TC, 1)


def _mm_kernel(x, W, a_s, a_d):
    h, asrc, adst = pl.pallas_call(
        _mm_body,
        grid=(_GRID,),
        in_specs=[
            pl.BlockSpec((NB_TC, D), lambda i: (i, 0)),
            pl.BlockSpec((D, D), lambda i: (0, 0)),
            pl.BlockSpec((1, D), lambda i: (0, 0)),
            pl.BlockSpec((1, D), lambda i: (0, 0)),
        ],
        out_specs=[
            pl.BlockSpec((NB_TC, D), lambda i: (i, 0)),
            pl.BlockSpec((NB_TC, 1), lambda i: (i, 0)),
            pl.BlockSpec((NB_TC, 1), lambda i: (i, 0)),
        ],
        out_shape=[
            jax.ShapeDtypeStruct((N, D), jnp.float32),
            jax.ShapeDtypeStruct((N, 1), jnp.float32),
            jax.ShapeDtypeStruct((N, 1), jnp.float32),
        ],
    )(x, W, a_s.reshape(1, D), a_d.reshape(1, D))
    return h, asrc.reshape(N), adst.reshape(N)


def _make_post_kernel(first_layer):
    def body(outp_ref, h_ref, ps_ref, stats_ref, asrc_ref, adst_ref, v_ref,
             b_ref, g_ref, be_ref, o_ref):
        st = stats_ref[...]
        deg = st[:, 0]
        att = st[:, 1:5]
        if first_layer:
            ssum_e = st[:, 5]
        else:
            ssum_e = ps_ref[...][:, 0]
        asrc = asrc_ref[...][:, 0]
        adst = adst_ref[...][:, 0]
        le = (att * v_ref[...]).sum(axis=1) / jnp.maximum(deg, 1.0)
        alself = asrc + adst + le
        wself = jnp.exp(jnp.where(alself > 0, alself, 0.2 * alself))
        outp = outp_ref[...]
        num = outp[0] + outp[1] + wself[:, None] * h_ref[...]
        den = ssum_e + wself + 1e-16
        o = num / den[:, None] + b_ref[...]
        o = o * _INV * g_ref[...] + be_ref[...]
        if first_layer:
            o = jnp.maximum(o, 0.0)
        o_ref[...] = o

    return pl.pallas_call(
        body,
        grid=(_GRID,),
        in_specs=[
            pl.BlockSpec((2, NB_TC, D), lambda i: (0, i, 0)),
            pl.BlockSpec((NB_TC, D), lambda i: (i, 0)),
            pl.BlockSpec((NB_TC, 1), lambda i: (i, 0)),
            pl.BlockSpec((NB_TC, 6), lambda i: (i, 0)),
            pl.BlockSpec((NB_TC, 1), lambda i: (i, 0)),
            pl.BlockSpec((NB_TC, 1), lambda i: (i, 0)),
            pl.BlockSpec((1, 4), lambda i: (0, 0)),
            pl.BlockSpec((1, D), lambda i: (0, 0)),
            pl.BlockSpec((1, D), lambda i: (0, 0)),
            pl.BlockSpec((1, D), lambda i: (0, 0)),
        ],
        out_specs=[pl.BlockSpec((NB_TC, D), lambda i: (i, 0))],
        out_shape=[jax.ShapeDtypeStruct((N, D), jnp.float32)],
    )


_post_l1 = _make_post_kernel(True)
_post_l2 = _make_post_kernel(False)


def kernel(x, edge_index, edge_att, W1, a_s1, a_d1, We1, ae1, b1, g1, be1,
           W2, a_s2, a_d2, We2, ae2, b2, g2, be2):
    src = edge_index[0].astype(jnp.int32)
    dst = edge_index[1].astype(jnp.int32)
    att_flat = edge_att.T.reshape(-1)
    zero_n1 = jnp.zeros((N, 1), jnp.float32)

    def layer(h_in, W, a_s, a_d, We, ae, b, g, be, first, stats):
        h, asrc, adst = _mm_kernel(h_in, W, a_s, a_d)
        v = We @ ae
        pvec = jnp.concatenate([v, jnp.zeros((12,), jnp.float32)])
        ak = _att_l1 if first else _att_l2
        w, part = ak(src, dst, att_flat, asrc, adst, pvec)
        (outp,) = _agg_kernel(src, dst, w, h)
        nch = 6 if first else 1
        ps = part.reshape(NW, nch, N).sum(axis=0).T
        if first:
            stats = ps
            ps_in = zero_n1
        else:
            ps_in = ps
        pk = _post_l1 if first else _post_l2
        (o,) = pk(outp, h, ps_in, stats, asrc.reshape(N, 1),
                  adst.reshape(N, 1), v.reshape(1, 4), b.reshape(1, D),
                  g.reshape(1, D), be.reshape(1, D))
        return o, stats

    h2, stats = layer(x, W1, a_s1, a_d1, We1, ae1, b1, g1, be1, True, None)
    out, _ = layer(h2, W2, a_s2, a_d2, We2, ae2, b2, g2, be2, False, stats)
    return out


# speedup vs baseline: 1.0367x; 1.0367x over previous
import functools

import jax
import jax.numpy as jnp
from jax import lax
from jax.experimental import pallas as pl
from jax.experimental.pallas import tpu as pltpu
from jax.experimental.pallas import tpu_sc as plsc

N = 10000
E = 320000
D = 128
EPS = 1e-5

NW = 32
EW = E // NW
CH = 2000
VPC = CH // 16
NCH = EW // CH
ZR = 16

_MESH = plsc.VectorSubcoreMesh(core_axis_name="c", subcore_axis_name="s")
_PARAMS = pltpu.CompilerParams(needs_layout_passes=False)


def _bcast_lane(vec, j):
    return jnp.take_along_axis(vec, jnp.full((16,), j, jnp.int32), axis=0)


def _make_attention_kernel(with_att):
    lacc = 6 * N if with_att else N

    def body(src_h, dst_h, att_h, asrc_h, adst_h, pv_h,
             w_h, part_h,
             asrc_v, adst_v, pv_v, acc_v, srcb, dstb, attb, wb):
        c = lax.axis_index("c")
        s = lax.axis_index("s")
        wid = s * 2 + c
        pltpu.sync_copy(asrc_h, asrc_v)
        pltpu.sync_copy(adst_h, adst_v)
        pltpu.sync_copy(pv_h, pv_v)
        pv = pv_v[...]
        v0 = _bcast_lane(pv, 0)
        v1 = _bcast_lane(pv, 1)
        v2 = _bcast_lane(pv, 2)
        v3 = _bcast_lane(pv, 3)
        zero16 = jnp.zeros((16,), jnp.float32)
        ones16 = jnp.ones((16,), jnp.float32)

        def zacc(i, carry):
            acc_v[pl.ds(i * 16, 16)] = zero16
            return carry
        lax.fori_loop(0, lacc // 16, zacc, 0)

        ebase = wid * EW

        def chunk(ci, carry):
            base = ebase + ci * CH
            pltpu.sync_copy(src_h.at[pl.ds(base, CH)], srcb)
            pltpu.sync_copy(dst_h.at[pl.ds(base, CH)], dstb)
            for ch in range(4):
                pltpu.sync_copy(att_h.at[pl.ds(ch * E + base, CH)],
                                attb.at[pl.ds(ch * CH, CH)])

            def ebody(v, carry2):
                o = v * 16
                sv = srcb[pl.ds(o, 16)]
                dv = dstb[pl.ds(o, 16)]
                a0 = attb[pl.ds(0 * CH + o, 16)]
                a1 = attb[pl.ds(1 * CH + o, 16)]
                a2 = attb[pl.ds(2 * CH + o, 16)]
                a3 = attb[pl.ds(3 * CH + o, 16)]
                ae = a0 * v0 + a1 * v1 + a2 * v2 + a3 * v3
                asg = plsc.load_gather(asrc_v, [sv])
                adg = plsc.load_gather(adst_v, [dv])
                al = asg + adg + ae
                al = jnp.where(al > 0, al, al * 0.2)
                wv = jnp.exp(al)
                wb[pl.ds(o, 16)] = wv
                if with_att:
                    plsc.addupdate_scatter(acc_v, [dv], ones16)
                    plsc.addupdate_scatter(acc_v, [dv + N], a0)
                    plsc.addupdate_scatter(acc_v, [dv + 2 * N], a1)
                    plsc.addupdate_scatter(acc_v, [dv + 3 * N], a2)
                    plsc.addupdate_scatter(acc_v, [dv + 4 * N], a3)
                    plsc.addupdate_scatter(acc_v, [dv + 5 * N], wv)
                else:
                    plsc.addupdate_scatter(acc_v, [dv], wv)
                return carry2
            lax.fori_loop(0, VPC, ebody, 0)
            pltpu.sync_copy(wb, w_h.at[pl.ds(base, CH)])
            return carry
        lax.fori_loop(0, NCH, chunk, 0)

        pltpu.sync_copy(acc_v, part_h.at[pl.ds(wid * lacc, lacc)])

    return pl.kernel(
        body,
        out_type=[jax.ShapeDtypeStruct((E,), jnp.float32),
                  jax.ShapeDtypeStruct((NW * lacc,), jnp.float32)],
        mesh=_MESH,
        scratch_types=[
            pltpu.VMEM((N,), jnp.float32),
            pltpu.VMEM((N,), jnp.float32),
            pltpu.VMEM((16,), jnp.float32),
            pltpu.VMEM((lacc,), jnp.float32),
            pltpu.VMEM((CH,), jnp.int32),
            pltpu.VMEM((CH,), jnp.int32),
            pltpu.VMEM((4 * CH,), jnp.float32),
            pltpu.VMEM((CH,), jnp.float32),
        ],
        compiler_params=_PARAMS,
    )


NB = 5
GROUPS = VPC // NB
ACH = EW
AVPC = ACH // 16
AGROUPS = AVPC // NB


def _agg_body(src_h, dst_h, w_h, h_h,
              outp_h,
              srcb, dstb, wb, gb, sb, shared_acc,
              gs0, gs1, gs2, gs3, gs4, ss0, ss1, ss2, ss3, ss4):
    gsems = (gs0, gs1, gs2, gs3, gs4)
    ssems = (ss0, ss1, ss2, ss3, ss4)
    c = lax.axis_index("c")
    s = lax.axis_index("s")
    wid = s * 2 + c
    zero16 = jnp.zeros((16,), jnp.float32)
    for r in range(ZR):
        for k in range(D // 16):
            sb[0, r, pl.ds(k * 16, 16)] = zero16
    zbase = s * 624

    def zsh(i, carry):
        pltpu.sync_copy(sb.at[0], shared_acc.at[pl.ds(zbase + i * ZR, ZR)])
        return carry
    lax.fori_loop(0, 624 // ZR, zsh, 0)

    @pl.when(s == 15)
    def _():
        pltpu.sync_copy(sb.at[0], shared_acc.at[pl.ds(624 * 16, ZR)])
    plsc.subcore_barrier()

    base = wid * EW
    pltpu.sync_copy(src_h.at[pl.ds(base, ACH)], srcb)
    pltpu.sync_copy(dst_h.at[pl.ds(base, ACH)], dstb)
    pltpu.sync_copy(w_h.at[pl.ds(base, ACH)], wb)
    for b in range(NB):
        svb = srcb[pl.ds(b * 16, 16)]
        pltpu.make_async_copy(h_h.at[svb], gb.at[b], gsems[b]).start()

    def group(g, carry2):
        for b in range(NB):
            v = g * NB + b
            o = v * 16
            dv = dstb[pl.ds(o, 16)]
            wv = wb[pl.ds(o, 16)]

            @pl.when(g > 0)
            def _wait_scatter():
                pltpu.make_async_copy(
                    sb.at[b], shared_acc.at[dv], ssems[b]).wait()
            pltpu.make_async_copy(h_h.at[dv], gb.at[b], gsems[b]).wait()
            for j in range(16):
                wj = _bcast_lane(wv, j)
                for k in range(D // 16):
                    sb[b, j, pl.ds(k * 16, 16)] = (
                        gb[b, j, pl.ds(k * 16, 16)] * wj)
            pltpu.make_async_copy(
                sb.at[b], shared_acc.at[dv], ssems[b]).start(add=True)

            @pl.when(g < AGROUPS - 1)
            def _issue_gather():
                svn = srcb[pl.ds((v + NB) * 16, 16)]
                pltpu.make_async_copy(h_h.at[svn], gb.at[b], gsems[b]).start()
        return carry2
    lax.fori_loop(0, AGROUPS, group, 0)
    dv0 = dstb[pl.ds(0, 16)]
    for b in range(NB):
        pltpu.make_async_copy(sb.at[b], shared_acc.at[dv0], ssems[b]).wait()

    plsc.subcore_barrier()

    @pl.when(s == 0)
    def _():
        pltpu.sync_copy(shared_acc, outp_h.at[c])


_agg_kernel = pl.kernel(
    _agg_body,
    out_type=[jax.ShapeDtypeStruct((2, N, D), jnp.float32)],
    mesh=_MESH,
    scratch_types=[
        pltpu.VMEM((ACH,), jnp.int32),
        pltpu.VMEM((ACH,), jnp.int32),
        pltpu.VMEM((ACH,), jnp.float32),
        pltpu.VMEM((NB, 16, D), jnp.float32),
        pltpu.VMEM((NB, 16, D), jnp.float32),
        pltpu.VMEM_SHARED((N, D), jnp.float32),
    ] + [pltpu.SemaphoreType.DMA] * 10,
    compiler_params=_PARAMS,
)

_att_l1 = _make_attention_kernel(True)
_att_l2 = _make_attention_kernel(False)


NB_TC = 2000
_GRID = N // NB_TC
_INV = float(1.0 / (1.0 + EPS) ** 0.5)


def _mm_body(x_ref, w_ref, as_ref, ad_ref, h_ref, asrc_ref, adst_ref):
    h = x_ref[...] @ w_ref[...]
    h_ref[...] = h
    asrc_ref[...] = (h * as_ref[...]).sum(axis=1).reshape(NB_TC, 1)
    adst_ref[...] = (h * ad_ref[...]).sum(axis=1).reshape(NB_TC, 1)


def _mm_kernel(x, W, a_s, a_d):
    h, asrc, adst = pl.pallas_call(
        _mm_body,
        grid=(_GRID,),
        in_specs=[
            pl.BlockSpec((NB_TC, D), lambda i: (i, 0)),
            pl.BlockSpec((D, D), lambda i: (0, 0)),
            pl.BlockSpec((1, D), lambda i: (0, 0)),
            pl.BlockSpec((1, D), lambda i: (0, 0)),
        ],
        out_specs=[
            pl.BlockSpec((NB_TC, D), lambda i: (i, 0)),
            pl.BlockSpec((NB_TC, 1), lambda i: (i, 0)),
            pl.BlockSpec((NB_TC, 1), lambda i: (i, 0)),
        ],
        out_shape=[
            jax.ShapeDtypeStruct((N, D), jnp.float32),
            jax.ShapeDtypeStruct((N, 1), jnp.float32),
            jax.ShapeDtypeStruct((N, 1), jnp.float32),
        ],
    )(x, W, a_s.reshape(1, D), a_d.reshape(1, D))
    return h, asrc.reshape(N), adst.reshape(N)


def kernel(x, edge_index, edge_att, W1, a_s1, a_d1, We1, ae1, b1, g1, be1,
           W2, a_s2, a_d2, We2, ae2, b2, g2, be2):
    src = edge_index[0].astype(jnp.int32)
    dst = edge_index[1].astype(jnp.int32)
    att_flat = edge_att.T.reshape(-1)
    inv = 1.0 / jnp.sqrt(1.0 + EPS)

    def layer(h_in, W, a_s, a_d, We, ae, b, first, att_stats):
        h, asrc, adst = _mm_kernel(h_in, W, a_s, a_d)
        v = We @ ae
        pvec = jnp.concatenate([v, jnp.zeros((12,), jnp.float32)])
        ak = _att_l1 if first else _att_l2
        w, part = ak(src, dst, att_flat, asrc, adst, pvec)
        lacc = 6 * N if first else N
        ps = part.reshape(NW, lacc).sum(axis=0)
        (outp,) = _agg_kernel(src, dst, w, h)
        if first:
            deg = ps[0:N]
            att_s = ps[N:5 * N].reshape(4, N)
            ssum_e = ps[5 * N:6 * N]
            att_stats = (deg, att_s)
        else:
            deg, att_s = att_stats
            ssum_e = ps
        le = (att_s * v[:, None]).sum(0) / jnp.clip(deg, 1.0)
        alself = asrc + adst + le
        wself = jnp.exp(jnp.where(alself > 0, alself, 0.2 * alself))
        num = outp[0] + outp[1] + wself[:, None] * h
        den = ssum_e + wself + 1e-16
        return num / den[:, None] + b, att_stats

    o1, stats = layer(x, W1, a_s1, a_d1, We1, ae1, b1, True, None)
    h2 = jax.nn.relu(o1 * inv * g1 + be1)
    o2, _ = layer(h2, W2, a_s2, a_d2, We2, ae2, b2, False, stats)
    return o2 * inv * g2 + be2


# speedup vs baseline: 1.0815x; 1.0432x over previous
import functools

import jax
import jax.numpy as jnp
from jax import lax
from jax.experimental import pallas as pl
from jax.experimental.pallas import tpu as pltpu
from jax.experimental.pallas import tpu_sc as plsc

N = 10000
E = 320000
D = 128
EPS = 1e-5

NW = 32
EW = E // NW
CH = 2000
VPC = CH // 16
NCH = EW // CH
ZR = 16

_MESH = plsc.VectorSubcoreMesh(core_axis_name="c", subcore_axis_name="s")
_PARAMS = pltpu.CompilerParams(needs_layout_passes=False)


def _bcast_lane(vec, j):
    return jnp.take_along_axis(vec, jnp.full((16,), j, jnp.int32), axis=0)


def _make_attention_kernel(with_att):
    lacc = 6 * N if with_att else N

    def body(src_h, dst_h, att_h, asrc_h, adst_h, pv_h,
             w_h, part_h,
             asrc_v, adst_v, pv_v, acc_v, srcb, dstb, attb, wb):
        c = lax.axis_index("c")
        s = lax.axis_index("s")
        wid = s * 2 + c
        pltpu.sync_copy(asrc_h, asrc_v)
        pltpu.sync_copy(adst_h, adst_v)
        pltpu.sync_copy(pv_h, pv_v)
        pv = pv_v[...]
        v0 = _bcast_lane(pv, 0)
        v1 = _bcast_lane(pv, 1)
        v2 = _bcast_lane(pv, 2)
        v3 = _bcast_lane(pv, 3)
        zero16 = jnp.zeros((16,), jnp.float32)
        ones16 = jnp.ones((16,), jnp.float32)

        def zacc(i, carry):
            acc_v[pl.ds(i * 16, 16)] = zero16
            return carry
        lax.fori_loop(0, lacc // 16, zacc, 0)

        ebase = wid * EW

        def chunk(ci, carry):
            base = ebase + ci * CH
            pltpu.sync_copy(src_h.at[pl.ds(base, CH)], srcb)
            pltpu.sync_copy(dst_h.at[pl.ds(base, CH)], dstb)
            for ch in range(4):
                pltpu.sync_copy(att_h.at[pl.ds(ch * E + base, CH)],
                                attb.at[pl.ds(ch * CH, CH)])

            def ebody(v, carry2):
                o = v * 16
                sv = srcb[pl.ds(o, 16)]
                dv = dstb[pl.ds(o, 16)]
                a0 = attb[pl.ds(0 * CH + o, 16)]
                a1 = attb[pl.ds(1 * CH + o, 16)]
                a2 = attb[pl.ds(2 * CH + o, 16)]
                a3 = attb[pl.ds(3 * CH + o, 16)]
                ae = a0 * v0 + a1 * v1 + a2 * v2 + a3 * v3
                asg = plsc.load_gather(asrc_v, [sv])
                adg = plsc.load_gather(adst_v, [dv])
                al = asg + adg + ae
                al = jnp.where(al > 0, al, al * 0.2)
                wv = jnp.exp(al)
                wb[pl.ds(o, 16)] = wv
                if with_att:
                    plsc.addupdate_scatter(acc_v, [dv], ones16)
                    plsc.addupdate_scatter(acc_v, [dv + N], a0)
                    plsc.addupdate_scatter(acc_v, [dv + 2 * N], a1)
                    plsc.addupdate_scatter(acc_v, [dv + 3 * N], a2)
                    plsc.addupdate_scatter(acc_v, [dv + 4 * N], a3)
                    plsc.addupdate_scatter(acc_v, [dv + 5 * N], wv)
                else:
                    plsc.addupdate_scatter(acc_v, [dv], wv)
                return carry2
            lax.fori_loop(0, VPC, ebody, 0)
            pltpu.sync_copy(wb, w_h.at[pl.ds(base, CH)])
            return carry
        lax.fori_loop(0, NCH, chunk, 0)

        pltpu.sync_copy(acc_v, part_h.at[pl.ds(wid * lacc, lacc)])

    return pl.kernel(
        body,
        out_type=[jax.ShapeDtypeStruct((E,), jnp.float32),
                  jax.ShapeDtypeStruct((NW * lacc,), jnp.float32)],
        mesh=_MESH,
        scratch_types=[
            pltpu.VMEM((N,), jnp.float32),
            pltpu.VMEM((N,), jnp.float32),
            pltpu.VMEM((16,), jnp.float32),
            pltpu.VMEM((lacc,), jnp.float32),
            pltpu.VMEM((CH,), jnp.int32),
            pltpu.VMEM((CH,), jnp.int32),
            pltpu.VMEM((4 * CH,), jnp.float32),
            pltpu.VMEM((CH,), jnp.float32),
        ],
        compiler_params=_PARAMS,
    )


NB = 5
GROUPS = VPC // NB
ACH = EW
AVPC = ACH // 16
AGROUPS = AVPC // NB


def _agg_body(src_h, dst_h, w_h, h_h,
              outp_h,
              srcb, dstb, wb, gb, sb, shared_acc,
              gs0, gs1, gs2, gs3, gs4, ss0, ss1, ss2, ss3, ss4):
    gsems = (gs0, gs1, gs2, gs3, gs4)
    ssems = (ss0, ss1, ss2, ss3, ss4)
    c = lax.axis_index("c")
    s = lax.axis_index("s")
    wid = s * 2 + c
    zero16 = jnp.zeros((16,), jnp.float32)
    for r in range(ZR):
        for k in range(D // 16):
            sb[0, r, pl.ds(k * 16, 16)] = zero16
    zbase = s * 624

    def zsh(i, carry):
        pltpu.sync_copy(sb.at[0], shared_acc.at[pl.ds(zbase + i * ZR, ZR)])
        return carry
    lax.fori_loop(0, 624 // ZR, zsh, 0)

    @pl.when(s == 15)
    def _():
        pltpu.sync_copy(sb.at[0], shared_acc.at[pl.ds(624 * 16, ZR)])
    plsc.subcore_barrier()

    base = wid * EW
    pltpu.sync_copy(src_h.at[pl.ds(base, ACH)], srcb)
    pltpu.sync_copy(dst_h.at[pl.ds(base, ACH)], dstb)
    pltpu.sync_copy(w_h.at[pl.ds(base, ACH)], wb)
    for b in range(NB):
        svb = srcb[pl.ds(b * 16, 16)]
        pltpu.make_async_copy(h_h.at[svb], gb.at[b], gsems[b]).start()

    def group(g, carry2):
        for b in range(NB):
            v = g * NB + b
            o = v * 16
            dv = dstb[pl.ds(o, 16)]
            wv = wb[pl.ds(o, 16)]

            @pl.when(g > 0)
            def _wait_scatter():
                pltpu.make_async_copy(
                    sb.at[b], shared_acc.at[dv], ssems[b]).wait()
            pltpu.make_async_copy(h_h.at[dv], gb.at[b], gsems[b]).wait()
            for j in range(16):
                wj = _bcast_lane(wv, j)
                for k in range(D // 16):
                    sb[b, j, pl.ds(k * 16, 16)] = (
                        gb[b, j, pl.ds(k * 16, 16)] * wj)
            pltpu.make_async_copy(
                sb.at[b], shared_acc.at[dv], ssems[b]).start(add=True)

            @pl.when(g < AGROUPS - 1)
            def _issue_gather():
                svn = srcb[pl.ds((v + NB) * 16, 16)]
                pltpu.make_async_copy(h_h.at[svn], gb.at[b], gsems[b]).start()
        return carry2
    lax.fori_loop(0, AGROUPS, group, 0)
    dv0 = dstb[pl.ds(0, 16)]
    for b in range(NB):
        pltpu.make_async_copy(sb.at[b], shared_acc.at[dv0], ssems[b]).wait()

    plsc.subcore_barrier()

    @pl.when(s == 0)
    def _():
        pltpu.sync_copy(shared_acc, outp_h.at[c])


_agg_kernel = pl.kernel(
    _agg_body,
    out_type=[jax.ShapeDtypeStruct((2, N, D), jnp.float32)],
    mesh=_MESH,
    scratch_types=[
        pltpu.VMEM((ACH,), jnp.int32),
        pltpu.VMEM((ACH,), jnp.int32),
        pltpu.VMEM((ACH,), jnp.float32),
        pltpu.VMEM((NB, 16, D), jnp.float32),
        pltpu.VMEM((NB, 16, D), jnp.float32),
        pltpu.VMEM_SHARED((N, D), jnp.float32),
    ] + [pltpu.SemaphoreType.DMA] * 10,
    compiler_params=_PARAMS,
)

_att_l1 = _make_attention_kernel(True)
_att_l2 = _make_attention_kernel(False)


def _dense_body(x_ref, w_ref, o_ref):
    o_ref[...] = x_ref[...] @ w_ref[...]


def _matmul(x, w):
    return pl.pallas_call(
        _dense_body,
        out_shape=jax.ShapeDtypeStruct((x.shape[0], w.shape[1]), jnp.float32),
    )(x, w)


def kernel(x, edge_index, edge_att, W1, a_s1, a_d1, We1, ae1, b1, g1, be1,
           W2, a_s2, a_d2, We2, ae2, b2, g2, be2):
    src = edge_index[0].astype(jnp.int32)
    dst = edge_index[1].astype(jnp.int32)
    att_flat = edge_att.T.reshape(-1)
    inv = 1.0 / jnp.sqrt(1.0 + EPS)

    def layer(h_in, W, a_s, a_d, We, ae, b, first, att_stats):
        h = _matmul(h_in, W)
        asrc = h @ a_s
        adst = h @ a_d
        v = We @ ae
        pvec = jnp.concatenate([v, jnp.zeros((12,), jnp.float32)])
        ak = _att_l1 if first else _att_l2
        w, part = ak(src, dst, att_flat, asrc, adst, pvec)
        lacc = 6 * N if first else N
        ps = part.reshape(NW, lacc).sum(axis=0)
        (outp,) = _agg_kernel(src, dst, w, h)
        if first:
            deg = ps[0:N]
            att_s = ps[N:5 * N].reshape(4, N)
            ssum_e = ps[5 * N:6 * N]
            att_stats = (deg, att_s)
        else:
            deg, att_s = att_stats
            ssum_e = ps
        le = (att_s * v[:, None]).sum(0) / jnp.clip(deg, 1.0)
        alself = asrc + adst + le
        wself = jnp.exp(jnp.where(alself > 0, alself, 0.2 * alself))
        num = outp[0] + outp[1] + wself[:, None] * h
        den = ssum_e + wself + 1e-16
        return num / den[:, None] + b, att_stats

    o1, stats = layer(x, W1, a_s1, a_d1, We1, ae1, b1, True, None)
    h2 = jax.nn.relu(o1 * inv * g1 + be1)
    o2, _ = layer(h2, W2, a_s2, a_d2, We2, ae2, b2, False, stats)
    return o2 * inv * g2 + be2


# speedup vs baseline: 1.1163x; 1.0322x over previous
import functools

import jax
import jax.numpy as jnp
from jax import lax
from jax.experimental import pallas as pl
from jax.experimental.pallas import tpu as pltpu
from jax.experimental.pallas import tpu_sc as plsc

N = 10000
E = 320000
D = 128
EPS = 1e-5

NW = 32
EW = E // NW
CH = 2000
VPC = CH // 16
NCH = EW // CH
ZR = 16

_MESH = plsc.VectorSubcoreMesh(core_axis_name="c", subcore_axis_name="s")
_PARAMS = pltpu.CompilerParams(needs_layout_passes=False)


def _bcast_lane(vec, j):
    return jnp.take_along_axis(vec, jnp.full((16,), j, jnp.int32), axis=0)


def _make_attention_kernel(with_att):
    lacc = 6 * N if with_att else N
    cch = CH if with_att else EW
    ncch = EW // cch
    vpcc = cch // 16

    def body(src_h, dst_h, att_h, asrc_h, adst_h, pv_h,
             w_h, part_h,
             asrc_v, adst_v, pv_v, acc_v, srcb, dstb, attb, wb):
        c = lax.axis_index("c")
        s = lax.axis_index("s")
        wid = s * 2 + c
        pltpu.sync_copy(asrc_h, asrc_v)
        pltpu.sync_copy(adst_h, adst_v)
        pltpu.sync_copy(pv_h, pv_v)
        pv = pv_v[...]
        v0 = _bcast_lane(pv, 0)
        v1 = _bcast_lane(pv, 1)
        v2 = _bcast_lane(pv, 2)
        v3 = _bcast_lane(pv, 3)
        zero16 = jnp.zeros((16,), jnp.float32)
        ones16 = jnp.ones((16,), jnp.float32)

        def zacc(i, carry):
            acc_v[pl.ds(i * 16, 16)] = zero16
            return carry
        lax.fori_loop(0, lacc // 16, zacc, 0)

        ebase = wid * EW

        def chunk(ci, carry):
            base = ebase + ci * cch
            pltpu.sync_copy(src_h.at[pl.ds(base, cch)], srcb)
            pltpu.sync_copy(dst_h.at[pl.ds(base, cch)], dstb)
            for ch in range(4):
                pltpu.sync_copy(att_h.at[pl.ds(ch * E + base, cch)],
                                attb.at[pl.ds(ch * cch, cch)])

            def ebody(v, carry2):
                o = v * 16
                sv = srcb[pl.ds(o, 16)]
                dv = dstb[pl.ds(o, 16)]
                a0 = attb[pl.ds(0 * cch + o, 16)]
                a1 = attb[pl.ds(1 * cch + o, 16)]
                a2 = attb[pl.ds(2 * cch + o, 16)]
                a3 = attb[pl.ds(3 * cch + o, 16)]
                ae = a0 * v0 + a1 * v1 + a2 * v2 + a3 * v3
                asg = plsc.load_gather(asrc_v, [sv])
                adg = plsc.load_gather(adst_v, [dv])
                al = asg + adg + ae
                al = jnp.where(al > 0, al, al * 0.2)
                wv = jnp.exp(al)
                wb[pl.ds(o, 16)] = wv
                if with_att:
                    plsc.addupdate_scatter(acc_v, [dv], ones16)
                    plsc.addupdate_scatter(acc_v, [dv + N], a0)
                    plsc.addupdate_scatter(acc_v, [dv + 2 * N], a1)
                    plsc.addupdate_scatter(acc_v, [dv + 3 * N], a2)
                    plsc.addupdate_scatter(acc_v, [dv + 4 * N], a3)
                    plsc.addupdate_scatter(acc_v, [dv + 5 * N], wv)
                else:
                    plsc.addupdate_scatter(acc_v, [dv], wv)
                return carry2
            lax.fori_loop(0, vpcc, ebody, 0)
            pltpu.sync_copy(wb, w_h.at[pl.ds(base, cch)])
            return carry
        lax.fori_loop(0, ncch, chunk, 0)

        pltpu.sync_copy(acc_v, part_h.at[pl.ds(wid * lacc, lacc)])

    return pl.kernel(
        body,
        out_type=[jax.ShapeDtypeStruct((E,), jnp.float32),
                  jax.ShapeDtypeStruct((NW * lacc,), jnp.float32)],
        mesh=_MESH,
        scratch_types=[
            pltpu.VMEM((N,), jnp.float32),
            pltpu.VMEM((N,), jnp.float32),
            pltpu.VMEM((16,), jnp.float32),
            pltpu.VMEM((lacc,), jnp.float32),
            pltpu.VMEM((cch,), jnp.int32),
            pltpu.VMEM((cch,), jnp.int32),
            pltpu.VMEM((4 * cch,), jnp.float32),
            pltpu.VMEM((cch,), jnp.float32),
        ],
        compiler_params=_PARAMS,
    )


NB = 5
GROUPS = VPC // NB
ACH = EW
AVPC = ACH // 16
AGROUPS = AVPC // NB


def _agg_body(src_h, dst_h, w_h, h_h,
              outp_h,
              srcb, dstb, wb, gb, sb, shared_acc,
              gs0, gs1, gs2, gs3, gs4, ss0, ss1, ss2, ss3, ss4):
    gsems = (gs0, gs1, gs2, gs3, gs4)
    ssems = (ss0, ss1, ss2, ss3, ss4)
    c = lax.axis_index("c")
    s = lax.axis_index("s")
    wid = s * 2 + c
    zero16 = jnp.zeros((16,), jnp.float32)
    for r in range(ZR):
        for k in range(D // 16):
            sb[0, r, pl.ds(k * 16, 16)] = zero16
    zbase = s * 624

    def zsh(i, carry):
        pltpu.sync_copy(sb.at[0], shared_acc.at[pl.ds(zbase + i * ZR, ZR)])
        return carry
    lax.fori_loop(0, 624 // ZR, zsh, 0)

    @pl.when(s == 15)
    def _():
        pltpu.sync_copy(sb.at[0], shared_acc.at[pl.ds(624 * 16, ZR)])
    plsc.subcore_barrier()

    base = wid * EW
    pltpu.sync_copy(src_h.at[pl.ds(base, ACH)], srcb)
    pltpu.sync_copy(dst_h.at[pl.ds(base, ACH)], dstb)
    pltpu.sync_copy(w_h.at[pl.ds(base, ACH)], wb)
    for b in range(NB):
        svb = srcb[pl.ds(b * 16, 16)]
        pltpu.make_async_copy(h_h.at[svb], gb.at[b], gsems[b]).start()

    def group(g, carry2):
        for b in range(NB):
            v = g * NB + b
            o = v * 16
            dv = dstb[pl.ds(o, 16)]
            wv = wb[pl.ds(o, 16)]

            @pl.when(g > 0)
            def _wait_scatter():
                pltpu.make_async_copy(
                    sb.at[b], shared_acc.at[dv], ssems[b]).wait()
            pltpu.make_async_copy(h_h.at[dv], gb.at[b], gsems[b]).wait()
            for j in range(16):
                wj = _bcast_lane(wv, j)
                for k in range(D // 16):
                    sb[b, j, pl.ds(k * 16, 16)] = (
                        gb[b, j, pl.ds(k * 16, 16)] * wj)
            pltpu.make_async_copy(
                sb.at[b], shared_acc.at[dv], ssems[b]).start(add=True)

            @pl.when(g < AGROUPS - 1)
            def _issue_gather():
                svn = srcb[pl.ds((v + NB) * 16, 16)]
                pltpu.make_async_copy(h_h.at[svn], gb.at[b], gsems[b]).start()
        return carry2
    lax.fori_loop(0, AGROUPS, group, 0)
    dv0 = dstb[pl.ds(0, 16)]
    for b in range(NB):
        pltpu.make_async_copy(sb.at[b], shared_acc.at[dv0], ssems[b]).wait()

    plsc.subcore_barrier()

    @pl.when(s == 0)
    def _():
        pltpu.sync_copy(shared_acc, outp_h.at[c])


_agg_kernel = pl.kernel(
    _agg_body,
    out_type=[jax.ShapeDtypeStruct((2, N, D), jnp.float32)],
    mesh=_MESH,
    scratch_types=[
        pltpu.VMEM((ACH,), jnp.int32),
        pltpu.VMEM((ACH,), jnp.int32),
        pltpu.VMEM((ACH,), jnp.float32),
        pltpu.VMEM((NB, 16, D), jnp.float32),
        pltpu.VMEM((NB, 16, D), jnp.float32),
        pltpu.VMEM_SHARED((N, D), jnp.float32),
    ] + [pltpu.SemaphoreType.DMA] * 10,
    compiler_params=_PARAMS,
)

_att_l1 = _make_attention_kernel(True)
_att_l2 = _make_attention_kernel(False)


def _dense_body(x_ref, w_ref, o_ref):
    o_ref[...] = x_ref[...] @ w_ref[...]


def _matmul(x, w):
    return pl.pallas_call(
        _dense_body,
        out_shape=jax.ShapeDtypeStruct((x.shape[0], w.shape[1]), jnp.float32),
    )(x, w)


def kernel(x, edge_index, edge_att, W1, a_s1, a_d1, We1, ae1, b1, g1, be1,
           W2, a_s2, a_d2, We2, ae2, b2, g2, be2):
    src = edge_index[0].astype(jnp.int32)
    dst = edge_index[1].astype(jnp.int32)
    att_flat = edge_att.T.reshape(-1)
    inv = 1.0 / jnp.sqrt(1.0 + EPS)

    def layer(h_in, W, a_s, a_d, We, ae, b, first, att_stats):
        h = _matmul(h_in, W)
        asrc = h @ a_s
        adst = h @ a_d
        v = We @ ae
        pvec = jnp.concatenate([v, jnp.zeros((12,), jnp.float32)])
        ak = _att_l1 if first else _att_l2
        w, part = ak(src, dst, att_flat, asrc, adst, pvec)
        lacc = 6 * N if first else N
        ps = part.reshape(NW, lacc).sum(axis=0)
        (outp,) = _agg_kernel(src, dst, w, h)
        if first:
            deg = ps[0:N]
            att_s = ps[N:5 * N].reshape(4, N)
            ssum_e = ps[5 * N:6 * N]
            att_stats = (deg, att_s)
        else:
            deg, att_s = att_stats
            ssum_e = ps
        le = (att_s * v[:, None]).sum(0) / jnp.clip(deg, 1.0)
        alself = asrc + adst + le
        wself = jnp.exp(jnp.where(alself > 0, alself, 0.2 * alself))
        num = outp[0] + outp[1] + wself[:, None] * h
        den = ssum_e + wself + 1e-16
        return num / den[:, None] + b, att_stats

    o1, stats = layer(x, W1, a_s1, a_d1, We1, ae1, b1, True, None)
    h2 = jax.nn.relu(o1 * inv * g1 + be1)
    o2, _ = layer(h2, W2, a_s2, a_d2, We2, ae2, b2, False, stats)
    return o2 * inv * g2 + be2


# speedup vs baseline: 1.1664x; 1.0448x over previous
import functools

import jax
import jax.numpy as jnp
from jax import lax
from jax.experimental import pallas as pl
from jax.experimental.pallas import tpu as pltpu
from jax.experimental.pallas import tpu_sc as plsc

N = 10000
E = 320000
D = 128
EPS = 1e-5

NW = 32
EW = E // NW
CH = 2000
VPC = CH // 16
NCH = EW // CH
ZR = 16

_MESH = plsc.VectorSubcoreMesh(core_axis_name="c", subcore_axis_name="s")
_PARAMS = pltpu.CompilerParams(needs_layout_passes=False)


def _bcast_lane(vec, j):
    return jnp.take_along_axis(vec, jnp.full((16,), j, jnp.int32), axis=0)


def _make_attention_kernel(with_att):
    lacc = 6 * N if with_att else N
    cch = CH if with_att else EW
    ncch = EW // cch
    vpcc = cch // 16

    def body(src_h, dst_h, att_h, asrc_h, adst_h, pv_h,
             w_h, part_h,
             asrc_v, adst_v, pv_v, acc_v, srcb, dstb, attb, wb, stsem):
        c = lax.axis_index("c")
        s = lax.axis_index("s")
        wid = s * 2 + c
        pltpu.sync_copy(asrc_h, asrc_v)
        pltpu.sync_copy(adst_h, adst_v)
        pltpu.sync_copy(pv_h, pv_v)
        pv = pv_v[...]
        v0 = _bcast_lane(pv, 0)
        v1 = _bcast_lane(pv, 1)
        v2 = _bcast_lane(pv, 2)
        v3 = _bcast_lane(pv, 3)
        zero16 = jnp.zeros((16,), jnp.float32)
        ones16 = jnp.ones((16,), jnp.float32)

        def zacc(i, carry):
            acc_v[pl.ds(i * 16, 16)] = zero16
            return carry
        lax.fori_loop(0, lacc // 16, zacc, 0)

        ebase = wid * EW

        def chunk(ci, carry):
            base = ebase + ci * cch
            descs = [
                pltpu.make_async_copy(src_h.at[pl.ds(base, cch)], srcb, stsem),
                pltpu.make_async_copy(dst_h.at[pl.ds(base, cch)], dstb, stsem),
            ] + [
                pltpu.make_async_copy(att_h.at[pl.ds(ch * E + base, cch)],
                                      attb.at[pl.ds(ch * cch, cch)], stsem)
                for ch in range(4)
            ]
            for d in descs:
                d.start()
            for d in descs:
                d.wait()

            def ebody(v, carry2):
                o = v * 16
                sv = srcb[pl.ds(o, 16)]
                dv = dstb[pl.ds(o, 16)]
                a0 = attb[pl.ds(0 * cch + o, 16)]
                a1 = attb[pl.ds(1 * cch + o, 16)]
                a2 = attb[pl.ds(2 * cch + o, 16)]
                a3 = attb[pl.ds(3 * cch + o, 16)]
                ae = a0 * v0 + a1 * v1 + a2 * v2 + a3 * v3
                asg = plsc.load_gather(asrc_v, [sv])
                adg = plsc.load_gather(adst_v, [dv])
                al = asg + adg + ae
                al = jnp.where(al > 0, al, al * 0.2)
                wv = jnp.exp(al)
                wb[pl.ds(o, 16)] = wv
                if with_att:
                    plsc.addupdate_scatter(acc_v, [dv], ones16)
                    plsc.addupdate_scatter(acc_v, [dv + N], a0)
                    plsc.addupdate_scatter(acc_v, [dv + 2 * N], a1)
                    plsc.addupdate_scatter(acc_v, [dv + 3 * N], a2)
                    plsc.addupdate_scatter(acc_v, [dv + 4 * N], a3)
                    plsc.addupdate_scatter(acc_v, [dv + 5 * N], wv)
                else:
                    plsc.addupdate_scatter(acc_v, [dv], wv)
                return carry2
            lax.fori_loop(0, vpcc, ebody, 0)
            pltpu.sync_copy(wb, w_h.at[pl.ds(base, cch)])
            return carry
        lax.fori_loop(0, ncch, chunk, 0)

        pltpu.sync_copy(acc_v, part_h.at[pl.ds(wid * lacc, lacc)])

    return pl.kernel(
        body,
        out_type=[jax.ShapeDtypeStruct((E,), jnp.float32),
                  jax.ShapeDtypeStruct((NW * lacc,), jnp.float32)],
        mesh=_MESH,
        scratch_types=[
            pltpu.VMEM((N,), jnp.float32),
            pltpu.VMEM((N,), jnp.float32),
            pltpu.VMEM((16,), jnp.float32),
            pltpu.VMEM((lacc,), jnp.float32),
            pltpu.VMEM((cch,), jnp.int32),
            pltpu.VMEM((cch,), jnp.int32),
            pltpu.VMEM((4 * cch,), jnp.float32),
            pltpu.VMEM((cch,), jnp.float32),
            pltpu.SemaphoreType.DMA,
        ],
        compiler_params=_PARAMS,
    )


NB = 5
GROUPS = VPC // NB
ACH = EW
AVPC = ACH // 16
AGROUPS = AVPC // NB


def _agg_body(src_h, dst_h, w_h, h_h,
              outp_h,
              srcb, dstb, wb, gb, sb, shared_acc,
              gs0, gs1, gs2, gs3, gs4, ss0, ss1, ss2, ss3, ss4):
    gsems = (gs0, gs1, gs2, gs3, gs4)
    ssems = (ss0, ss1, ss2, ss3, ss4)
    c = lax.axis_index("c")
    s = lax.axis_index("s")
    wid = s * 2 + c
    zero16 = jnp.zeros((16,), jnp.float32)
    for r in range(ZR):
        for k in range(D // 16):
            sb[0, r, pl.ds(k * 16, 16)] = zero16
    zbase = s * 624

    def zsh(i, carry):
        pltpu.sync_copy(sb.at[0], shared_acc.at[pl.ds(zbase + i * ZR, ZR)])
        return carry
    lax.fori_loop(0, 624 // ZR, zsh, 0)

    @pl.when(s == 15)
    def _():
        pltpu.sync_copy(sb.at[0], shared_acc.at[pl.ds(624 * 16, ZR)])
    plsc.subcore_barrier()

    base = wid * EW
    descs = [
        pltpu.make_async_copy(src_h.at[pl.ds(base, ACH)], srcb, gs0),
        pltpu.make_async_copy(dst_h.at[pl.ds(base, ACH)], dstb, gs0),
        pltpu.make_async_copy(w_h.at[pl.ds(base, ACH)], wb, gs0),
    ]
    for d in descs:
        d.start()
    for d in descs:
        d.wait()
    for b in range(NB):
        svb = srcb[pl.ds(b * 16, 16)]
        pltpu.make_async_copy(h_h.at[svb], gb.at[b], gsems[b]).start()

    def group(g, carry2):
        for b in range(NB):
            v = g * NB + b
            o = v * 16
            dv = dstb[pl.ds(o, 16)]
            wv = wb[pl.ds(o, 16)]

            @pl.when(g > 0)
            def _wait_scatter():
                pltpu.make_async_copy(
                    sb.at[b], shared_acc.at[dv], ssems[b]).wait()
            pltpu.make_async_copy(h_h.at[dv], gb.at[b], gsems[b]).wait()
            for j in range(16):
                wj = _bcast_lane(wv, j)
                for k in range(D // 16):
                    sb[b, j, pl.ds(k * 16, 16)] = (
                        gb[b, j, pl.ds(k * 16, 16)] * wj)
            pltpu.make_async_copy(
                sb.at[b], shared_acc.at[dv], ssems[b]).start(add=True)

            @pl.when(g < AGROUPS - 1)
            def _issue_gather():
                svn = srcb[pl.ds((v + NB) * 16, 16)]
                pltpu.make_async_copy(h_h.at[svn], gb.at[b], gsems[b]).start()
        return carry2
    lax.fori_loop(0, AGROUPS, group, 0)
    dv0 = dstb[pl.ds(0, 16)]
    for b in range(NB):
        pltpu.make_async_copy(sb.at[b], shared_acc.at[dv0], ssems[b]).wait()

    plsc.subcore_barrier()

    @pl.when(s == 0)
    def _():
        pltpu.sync_copy(shared_acc, outp_h.at[c])


_agg_kernel = pl.kernel(
    _agg_body,
    out_type=[jax.ShapeDtypeStruct((2, N, D), jnp.float32)],
    mesh=_MESH,
    scratch_types=[
        pltpu.VMEM((ACH,), jnp.int32),
        pltpu.VMEM((ACH,), jnp.int32),
        pltpu.VMEM((ACH,), jnp.float32),
        pltpu.VMEM((NB, 16, D), jnp.float32),
        pltpu.VMEM((NB, 16, D), jnp.float32),
        pltpu.VMEM_SHARED((N, D), jnp.float32),
    ] + [pltpu.SemaphoreType.DMA] * 10,
    compiler_params=_PARAMS,
)

_att_l1 = _make_attention_kernel(True)
_att_l2 = _make_attention_kernel(False)


def _dense_body(x_ref, w_ref, o_ref):
    o_ref[...] = x_ref[...] @ w_ref[...]


def _matmul(x, w):
    return pl.pallas_call(
        _dense_body,
        out_shape=jax.ShapeDtypeStruct((x.shape[0], w.shape[1]), jnp.float32),
    )(x, w)


def kernel(x, edge_index, edge_att, W1, a_s1, a_d1, We1, ae1, b1, g1, be1,
           W2, a_s2, a_d2, We2, ae2, b2, g2, be2):
    src = edge_index[0].astype(jnp.int32)
    dst = edge_index[1].astype(jnp.int32)
    att_flat = edge_att.T.reshape(-1)
    inv = 1.0 / jnp.sqrt(1.0 + EPS)

    def layer(h_in, W, a_s, a_d, We, ae, b, first, att_stats):
        h = _matmul(h_in, W)
        asrc = h @ a_s
        adst = h @ a_d
        v = We @ ae
        pvec = jnp.concatenate([v, jnp.zeros((12,), jnp.float32)])
        ak = _att_l1 if first else _att_l2
        w, part = ak(src, dst, att_flat, asrc, adst, pvec)
        lacc = 6 * N if first else N
        ps = part.reshape(NW, lacc).sum(axis=0)
        (outp,) = _agg_kernel(src, dst, w, h)
        if first:
            deg = ps[0:N]
            att_s = ps[N:5 * N].reshape(4, N)
            ssum_e = ps[5 * N:6 * N]
            att_stats = (deg, att_s)
        else:
            deg, att_s = att_stats
            ssum_e = ps
        le = (att_s * v[:, None]).sum(0) / jnp.clip(deg, 1.0)
        alself = asrc + adst + le
        wself = jnp.exp(jnp.where(alself > 0, alself, 0.2 * alself))
        num = outp[0] + outp[1] + wself[:, None] * h
        den = ssum_e + wself + 1e-16
        return num / den[:, None] + b, att_stats

    o1, stats = layer(x, W1, a_s1, a_d1, We1, ae1, b1, True, None)
    h2 = jax.nn.relu(o1 * inv * g1 + be1)
    o2, _ = layer(h2, W2, a_s2, a_d2, We2, ae2, b2, False, stats)
    return o2 * inv * g2 + be2
